# baseline (speedup 1.0000x reference)
import dataclasses
import functools

import jax
import jax.numpy as jnp
from jax import lax
from jax.experimental import pallas as pl
from jax.experimental.pallas import tpu as pltpu
from jax.experimental.pallas import tpu_sc as plsc

N = 10000
D = 128
E = 320000
NC, NS, L = 2, 16, 16
CH = 128
GS = 8
CPT = 80
NGT = 20
NG0 = 10
NG1 = NGT - NG0
E_PAD = NS * NGT * GS * CH
TRASH = N
R = 10240
RPT = R // NS
NBLK = 10
BR = R // NBLK


@functools.cache
def _sc_mesh():
    return plsc.VectorSubcoreMesh(
        core_axis_name="c", subcore_axis_name="s",
        num_cores=NC, num_subcores=NS)


def _deg_body(dst_hbm, out_hbm, didx, hist):
    c = lax.axis_index("c")
    s = lax.axis_index("s")
    w = c * NS + s
    pltpu.sync_copy(dst_hbm.at[c, s], didx)

    @pl.loop(0, R, step=L)
    def _zero(i):
        hist[pl.ds(i, L)] = jnp.zeros((L,), jnp.float32)

    ones = jnp.ones((L,), jnp.float32)

    @pl.loop(0, CPT)
    def _chunk(g):
        @pl.loop(0, CH // L)
        def _grp(j):
            idx16 = didx[g, pl.ds(j * L, L)]
            plsc.addupdate_scatter(hist, [idx16], ones)

    pltpu.sync_copy(hist, out_hbm.at[w])


def _spmm_body(x1_hbm, srcg_hbm, dst_hbm, out_hbm,
               sga, sgb, dga, dgb, buf_a, buf_b, zblk, acc,
               sem_a, sem_b, sem_i, sem_id, ssc_a, ssc_b):
    c = lax.axis_index("c")
    s = lax.axis_index("s")

    @pl.loop(0, L)
    def _zr(r):
        @pl.loop(0, D // L)
        def _zc(j):
            zblk[r, pl.ds(j * L, L)] = jnp.zeros((L,), jnp.float32)

    @pl.loop(0, RPT // L)
    def _za(t):
        pltpu.sync_copy(zblk, acc.at[pl.ds(s * RPT + t * L, L)])

    sgrps, dgrps = [sga, sgb], [dga, dgb]
    bufs, sems, ssems = [buf_a, buf_b], [sem_a, sem_b], [ssc_a, ssc_b]

    @pl.when(c == 0)
    def _core0():
        _pipeline(x1_hbm, srcg_hbm, dst_hbm, s, sgrps, dgrps, bufs, sems,
                  ssems, sem_i, sem_id, acc, 0, NG0)

    @pl.when(c == 1)
    def _core1():
        _pipeline(x1_hbm, srcg_hbm, dst_hbm, s, sgrps, dgrps, bufs, sems,
                  ssems, sem_i, sem_id, acc, NG0, NG1)

    plsc.subcore_barrier()

    @pl.loop(0, RPT // CH)
    def _wb(t):
        base = s * RPT + t * CH
        pltpu.sync_copy(acc.at[pl.ds(base, CH)],
                        out_hbm.at[c, pl.ds(base, CH)])


def _pipeline(x1_hbm, srcg_hbm, dstg_hbm, s, sgrps, dgrps, bufs, sems,
              ssems, sem_i, sem_id, acc, goff, ng):
    cpt = ng * GS
    pltpu.sync_copy(srcg_hbm.at[s, goff], sgrps[0])
    pltpu.sync_copy(dstg_hbm.at[s, goff], dgrps[0])
    if ng > 1:
        pltpu.async_copy(srcg_hbm.at[s, goff + 1], sgrps[1], sem_i)
    pltpu.async_copy(x1_hbm.at[sgrps[0].at[0]], bufs[0], sems[0])
    for g in range(cpt):
        grp, row = divmod(g, GS)
        if g + 1 < cpt:
            grp1, row1 = divmod(g + 1, GS)
            if row1 == 0:
                pltpu.make_async_copy(
                    srcg_hbm.at[s, goff + grp1], sgrps[grp1 % 2],
                    sem_i).wait()
            if g >= 1:
                pltpu.make_async_copy(
                    bufs[(g + 1) % 2],
                    acc.at[dgrps[((g - 1) // GS) % 2].at[(g - 1) % GS]],
                    ssems[(g + 1) % 2]).wait()
            pltpu.async_copy(
                x1_hbm.at[sgrps[grp1 % 2].at[row1]],
                bufs[(g + 1) % 2], sems[(g + 1) % 2])
        pltpu.make_async_copy(
            x1_hbm.at[sgrps[grp % 2].at[row]], bufs[g % 2], sems[g % 2]
        ).wait()
        if row == 0:
            if grp > 0:
                pltpu.make_async_copy(
                    dstg_hbm.at[s, goff + grp], dgrps[grp % 2],
                    sem_id).wait()
            if grp + 1 < ng:
                pltpu.async_copy(
                    dstg_hbm.at[s, goff + grp + 1], dgrps[(grp + 1) % 2],
                    sem_id)
        if g + 1 < cpt and (g + 1) % GS == 0 and (g + 1) // GS + 1 < ng:
            pltpu.async_copy(
                srcg_hbm.at[s, goff + (g + 1) // GS + 1],
                sgrps[((g + 1) // GS + 1) % 2], sem_i)
        pltpu.async_copy(bufs[g % 2], acc.at[dgrps[grp % 2].at[row]],
                         ssems[g % 2], add=True)
    pltpu.make_async_copy(
        bufs[(cpt - 2) % 2], acc.at[dgrps[((cpt - 2) // GS) % 2]
                                    .at[(cpt - 2) % GS]],
        ssems[(cpt - 2) % 2]).wait()
    pltpu.make_async_copy(
        bufs[(cpt - 1) % 2], acc.at[dgrps[((cpt - 1) // GS) % 2]
                                    .at[(cpt - 1) % GS]],
        ssems[(cpt - 1) % 2]).wait()


def _normx_body(hists_ref, x_ref, x1_ref):
    deg = jnp.sum(hists_ref[...], axis=0)
    norm = lax.rsqrt(1.0 + deg)[:, None]
    x1_ref[...] = norm * x_ref[...]


@functools.cache
def _sc_kernels():
    cp = pltpu.CompilerParams()
    if "needs_layout_passes" in pltpu.CompilerParams.__dataclass_fields__:
        cp = dataclasses.replace(cp, needs_layout_passes=False)
    deg_k = pl.kernel(
        _deg_body,
        out_type=jax.ShapeDtypeStruct((NC * NS, R), jnp.float32),
        mesh=_sc_mesh(),
        compiler_params=cp,
        scratch_types=[
            pltpu.VMEM((CPT, CH), jnp.int32),
            pltpu.VMEM((R,), jnp.float32),
        ],
    )
    spmm_k = pl.kernel(
        _spmm_body,
        out_type=jax.ShapeDtypeStruct((NC, R, D), jnp.float32),
        mesh=_sc_mesh(),
        scratch_types=[
            pltpu.VMEM((GS, CH), jnp.int32),
            pltpu.VMEM((GS, CH), jnp.int32),
            pltpu.VMEM((GS, CH), jnp.int32),
            pltpu.VMEM((GS, CH), jnp.int32),
            pltpu.VMEM((CH, D), jnp.float32),
            pltpu.VMEM((CH, D), jnp.float32),
            pltpu.VMEM((L, D), jnp.float32),
            pltpu.VMEM_SHARED((R, D), jnp.float32),
            pltpu.SemaphoreType.DMA,
            pltpu.SemaphoreType.DMA,
            pltpu.SemaphoreType.DMA,
            pltpu.SemaphoreType.DMA,
            pltpu.SemaphoreType.DMA,
            pltpu.SemaphoreType.DMA,
        ],
    )
    return deg_k, spmm_k


def _combine_body(hists_ref, aggs_ref, x_ref, o_ref):
    deg = jnp.sum(hists_ref[...], axis=0)
    norm = lax.rsqrt(1.0 + deg)[:, None]
    agg = aggs_ref[0] + aggs_ref[1]
    o_ref[...] = norm * agg + (norm * norm) * x_ref[...]


def kernel(x, edge_index):
    ei = edge_index.astype(jnp.int32)
    dst = ei[0]
    src = ei[1]
    pad = E_PAD - E
    dst_flat = jnp.concatenate([dst, jnp.full((pad,), TRASH, jnp.int32)])
    src_flat = jnp.concatenate([src, jnp.zeros((pad,), jnp.int32)])
    dst_deg = dst_flat.reshape(NC, NS, CPT, CH)
    dst_sp = dst_flat.reshape(NS, NGT, GS, CH)
    src_sp = src_flat.reshape(NS, NGT, GS, CH)

    deg_k, spmm_k = _sc_kernels()
    hists = deg_k(dst_deg)

    x1 = pl.pallas_call(
        _normx_body,
        grid=(NBLK,),
        in_specs=[
            pl.BlockSpec((NC * NS, BR), lambda i: (0, i)),
            pl.BlockSpec((BR, D), lambda i: (i, 0)),
        ],
        out_specs=pl.BlockSpec((BR, D), lambda i: (i, 0)),
        out_shape=jax.ShapeDtypeStruct((R, D), jnp.float32),
    )(hists, x)

    aggs = spmm_k(x1, src_sp, dst_sp)

    out = pl.pallas_call(
        _combine_body,
        grid=(NBLK,),
        in_specs=[
            pl.BlockSpec((NC * NS, BR), lambda i: (0, i)),
            pl.BlockSpec((NC, BR, D), lambda i: (0, i, 0)),
            pl.BlockSpec((BR, D), lambda i: (i, 0)),
        ],
        out_specs=pl.BlockSpec((BR, D), lambda i: (i, 0)),
        out_shape=jax.ShapeDtypeStruct((N, D), jnp.float32),
    )(hists, aggs, x)
    return out


# speedup vs baseline: 2.8690x; 2.8690x over previous
import dataclasses
import functools

import jax
import jax.numpy as jnp
from jax import lax
from jax.experimental import pallas as pl
from jax.experimental.pallas import tpu as pltpu
from jax.experimental.pallas import tpu_sc as plsc

N = 10000
D = 128
E = 320000
NC, NS, L = 2, 16, 16
CH = 128
GS = 8
CPT = 80
NGT = 20
NG0 = 10
NG1 = NGT - NG0
E_PAD = NS * NGT * GS * CH
TRASH = N
R = 10240
RPT = R // NS
NBLK = 10
BR = R // NBLK


@functools.cache
def _sc_mesh():
    return plsc.VectorSubcoreMesh(
        core_axis_name="c", subcore_axis_name="s",
        num_cores=NC, num_subcores=NS)


def _deg_body(dst_hbm, out_hbm, didx, hist):
    c = lax.axis_index("c")
    s = lax.axis_index("s")
    w = c * NS + s
    pltpu.sync_copy(dst_hbm.at[c, s], didx)

    @pl.loop(0, R, step=L)
    def _zero(i):
        hist[pl.ds(i, L)] = jnp.zeros((L,), jnp.float32)

    ones = jnp.ones((L,), jnp.float32)

    @pl.loop(0, CPT)
    def _chunk(g):
        @pl.loop(0, CH // L)
        def _grp(j):
            idx16 = didx[g, pl.ds(j * L, L)]
            plsc.addupdate_scatter(hist, [idx16], ones)

    pltpu.sync_copy(hist, out_hbm.at[w])


def _spmm_body(x1_hbm, srcg_hbm, dst_hbm, out_hbm,
               sga, sgb, dga, dgb, buf_a, buf_b, zblk, acc,
               sem_a, sem_b, sem_i, sem_id, ssc_a, ssc_b):
    c = lax.axis_index("c")
    s = lax.axis_index("s")

    @pl.loop(0, L)
    def _zr(r):
        @pl.loop(0, D // L)
        def _zc(j):
            zblk[r, pl.ds(j * L, L)] = jnp.zeros((L,), jnp.float32)

    @pl.loop(0, RPT // L)
    def _za(t):
        pltpu.sync_copy(zblk, acc.at[pl.ds(s * RPT + t * L, L)])

    sgrps, dgrps = [sga, sgb], [dga, dgb]
    bufs, sems, ssems = [buf_a, buf_b], [sem_a, sem_b], [ssc_a, ssc_b]

    @pl.when(c == 0)
    def _core0():
        _pipeline(x1_hbm, srcg_hbm, dst_hbm, s, sgrps, dgrps, bufs, sems,
                  ssems, sem_i, sem_id, acc, 0, NG0)

    @pl.when(c == 1)
    def _core1():
        _pipeline(x1_hbm, srcg_hbm, dst_hbm, s, sgrps, dgrps, bufs, sems,
                  ssems, sem_i, sem_id, acc, NG0, NG1)

    plsc.subcore_barrier()

    @pl.loop(0, RPT // CH)
    def _wb(t):
        base = s * RPT + t * CH
        pltpu.sync_copy(acc.at[pl.ds(base, CH)],
                        out_hbm.at[c, pl.ds(base, CH)])


def _pipeline(x1_hbm, srcg_hbm, dstg_hbm, s, sgrps, dgrps, bufs, sems,
              ssems, sem_i, sem_id, acc, goff, ng):
    cpt = ng * GS
    pltpu.sync_copy(srcg_hbm.at[s, goff], sgrps[0])
    pltpu.sync_copy(dstg_hbm.at[s, goff], dgrps[0])
    if ng > 1:
        pltpu.async_copy(srcg_hbm.at[s, goff + 1], sgrps[1], sem_i)
    pltpu.async_copy(x1_hbm.at[sgrps[0].at[0]], bufs[0], sems[0])
    for g in range(cpt):
        grp, row = divmod(g, GS)
        if g + 1 < cpt:
            grp1, row1 = divmod(g + 1, GS)
            if row1 == 0:
                pltpu.make_async_copy(
                    srcg_hbm.at[s, goff + grp1], sgrps[grp1 % 2],
                    sem_i).wait()
            if g >= 1:
                pltpu.make_async_copy(
                    bufs[(g + 1) % 2],
                    acc.at[dgrps[((g - 1) // GS) % 2].at[(g - 1) % GS]],
                    ssems[(g + 1) % 2]).wait()
            pltpu.async_copy(
                x1_hbm.at[sgrps[grp1 % 2].at[row1]],
                bufs[(g + 1) % 2], sems[(g + 1) % 2])
        pltpu.make_async_copy(
            x1_hbm.at[sgrps[grp % 2].at[row]], bufs[g % 2], sems[g % 2]
        ).wait()
        if row == 0:
            if grp > 0:
                pltpu.make_async_copy(
                    dstg_hbm.at[s, goff + grp], dgrps[grp % 2],
                    sem_id).wait()
            if grp + 1 < ng:
                pltpu.async_copy(
                    dstg_hbm.at[s, goff + grp + 1], dgrps[(grp + 1) % 2],
                    sem_id)
        if g + 1 < cpt and (g + 1) % GS == 0 and (g + 1) // GS + 1 < ng:
            pltpu.async_copy(
                srcg_hbm.at[s, goff + (g + 1) // GS + 1],
                sgrps[((g + 1) // GS + 1) % 2], sem_i)
        pltpu.async_copy(bufs[g % 2], acc.at[dgrps[grp % 2].at[row]],
                         ssems[g % 2], add=True)
    pltpu.make_async_copy(
        bufs[(cpt - 2) % 2], acc.at[dgrps[((cpt - 2) // GS) % 2]
                                    .at[(cpt - 2) % GS]],
        ssems[(cpt - 2) % 2]).wait()
    pltpu.make_async_copy(
        bufs[(cpt - 1) % 2], acc.at[dgrps[((cpt - 1) // GS) % 2]
                                    .at[(cpt - 1) % GS]],
        ssems[(cpt - 1) % 2]).wait()


def _normx_body(hists_ref, x_ref, x1_ref):
    deg = jnp.sum(hists_ref[...], axis=0)
    norm = lax.rsqrt(1.0 + deg)[:, None]
    x1_ref[...] = norm * x_ref[...]


@functools.cache
def _sc_kernels():
    cp = pltpu.CompilerParams()
    if "needs_layout_passes" in pltpu.CompilerParams.__dataclass_fields__:
        cp = dataclasses.replace(cp, needs_layout_passes=False)
    deg_k = pl.kernel(
        _deg_body,
        out_type=jax.ShapeDtypeStruct((NC * NS, R), jnp.float32),
        mesh=_sc_mesh(),
        compiler_params=cp,
        scratch_types=[
            pltpu.VMEM((CPT, CH), jnp.int32),
            pltpu.VMEM((R,), jnp.float32),
        ],
    )
    spmm_k = pl.kernel(
        _spmm_body,
        out_type=jax.ShapeDtypeStruct((NC, R, D), jnp.float32),
        mesh=_sc_mesh(),
        scratch_types=[
            pltpu.VMEM((GS, CH), jnp.int32),
            pltpu.VMEM((GS, CH), jnp.int32),
            pltpu.VMEM((GS, CH), jnp.int32),
            pltpu.VMEM((GS, CH), jnp.int32),
            pltpu.VMEM((CH, D), jnp.float32),
            pltpu.VMEM((CH, D), jnp.float32),
            pltpu.VMEM((L, D), jnp.float32),
            pltpu.VMEM_SHARED((R, D), jnp.float32),
            pltpu.SemaphoreType.DMA,
            pltpu.SemaphoreType.DMA,
            pltpu.SemaphoreType.DMA,
            pltpu.SemaphoreType.DMA,
            pltpu.SemaphoreType.DMA,
            pltpu.SemaphoreType.DMA,
        ],
    )
    return deg_k, spmm_k


def _combine_body(hists_ref, aggs_ref, x_ref, o_ref):
    deg = jnp.sum(hists_ref[...], axis=0)
    norm = lax.rsqrt(1.0 + deg)[:, None]
    agg = aggs_ref[0] + aggs_ref[1]
    o_ref[...] = norm * agg + (norm * norm) * x_ref[...]


def kernel(x, edge_index):
    ei = edge_index.astype(jnp.int32)
    dst = ei[0]
    src = ei[1]
    pad = E_PAD - E
    pad_iota = jnp.arange(pad, dtype=jnp.int32)
    dst_flat = jnp.concatenate([dst, TRASH + pad_iota % (R - TRASH)])
    src_flat = jnp.concatenate([src, pad_iota % N])
    dst_deg = dst_flat.reshape(NC, NS, CPT, CH)
    dst_sp = dst_flat.reshape(NS, NGT, GS, CH)
    src_sp = src_flat.reshape(NS, NGT, GS, CH)

    deg_k, spmm_k = _sc_kernels()
    hists = deg_k(dst_deg)

    x1 = pl.pallas_call(
        _normx_body,
        grid=(NBLK,),
        in_specs=[
            pl.BlockSpec((NC * NS, BR), lambda i: (0, i)),
            pl.BlockSpec((BR, D), lambda i: (i, 0)),
        ],
        out_specs=pl.BlockSpec((BR, D), lambda i: (i, 0)),
        out_shape=jax.ShapeDtypeStruct((R, D), jnp.float32),
    )(hists, x)

    aggs = spmm_k(x1, src_sp, dst_sp)

    out = pl.pallas_call(
        _combine_body,
        grid=(NBLK,),
        in_specs=[
            pl.BlockSpec((NC * NS, BR), lambda i: (0, i)),
            pl.BlockSpec((NC, BR, D), lambda i: (0, i, 0)),
            pl.BlockSpec((BR, D), lambda i: (i, 0)),
        ],
        out_specs=pl.BlockSpec((BR, D), lambda i: (i, 0)),
        out_shape=jax.ShapeDtypeStruct((N, D), jnp.float32),
    )(hists, aggs, x)
    return out


# speedup vs baseline: 2.8951x; 1.0091x over previous
import dataclasses
import functools

import jax
import jax.numpy as jnp
from jax import lax
from jax.experimental import pallas as pl
from jax.experimental.pallas import tpu as pltpu
from jax.experimental.pallas import tpu_sc as plsc

N = 10000
D = 128
E = 320000
NC, NS, L = 2, 16, 16
CH = 128
GS = 8
CPT = 80
NGT = 20
NG0 = 10
NG1 = NGT - NG0
E_PAD = NS * NGT * GS * CH
TRASH = N
R = 10240
RPT = R // NS
NBLK = 10
BR = R // NBLK


@functools.cache
def _sc_mesh():
    return plsc.VectorSubcoreMesh(
        core_axis_name="c", subcore_axis_name="s",
        num_cores=NC, num_subcores=NS)


def _deg_body(dst_hbm, out_hbm, didx, hist):
    c = lax.axis_index("c")
    s = lax.axis_index("s")
    w = c * NS + s
    pltpu.sync_copy(dst_hbm.at[c, s], didx)

    @pl.loop(0, R, step=L)
    def _zero(i):
        hist[pl.ds(i, L)] = jnp.zeros((L,), jnp.float32)

    ones = jnp.ones((L,), jnp.float32)

    @pl.loop(0, CPT)
    def _chunk(g):
        @pl.loop(0, CH // L)
        def _grp(j):
            idx16 = didx[g, pl.ds(j * L, L)]
            plsc.addupdate_scatter(hist, [idx16], ones)

    pltpu.sync_copy(hist, out_hbm.at[w])


def _spmm_body(x1_hbm, srcg_hbm, dst_hbm, out_hbm,
               sga, sgb, dga, dgb, buf_a, buf_b, zblk, acc,
               sem_a, sem_b, sem_i, sem_id, ssc_a, ssc_b):
    c = lax.axis_index("c")
    s = lax.axis_index("s")

    @pl.loop(0, L)
    def _zr(r):
        @pl.loop(0, D // L)
        def _zc(j):
            zblk[r, pl.ds(j * L, L)] = jnp.zeros((L,), jnp.float32)

    @pl.loop(0, RPT // L)
    def _za(t):
        pltpu.async_copy(zblk, acc.at[pl.ds(s * RPT + t * L, L)], sem_id)

    @pl.loop(0, RPT // L)
    def _zd(t):
        pltpu.make_async_copy(
            zblk, acc.at[pl.ds(s * RPT + t * L, L)], sem_id).wait()

    plsc.subcore_barrier()

    sgrps, dgrps = [sga, sgb], [dga, dgb]
    bufs, sems, ssems = [buf_a, buf_b], [sem_a, sem_b], [ssc_a, ssc_b]

    @pl.when(c == 0)
    def _core0():
        _pipeline(x1_hbm, srcg_hbm, dst_hbm, s, sgrps, dgrps, bufs, sems,
                  ssems, sem_i, sem_id, acc, 0, NG0)

    @pl.when(c == 1)
    def _core1():
        _pipeline(x1_hbm, srcg_hbm, dst_hbm, s, sgrps, dgrps, bufs, sems,
                  ssems, sem_i, sem_id, acc, NG0, NG1)

    plsc.subcore_barrier()

    @pl.loop(0, RPT // CH)
    def _wb(t):
        base = s * RPT + t * CH
        pltpu.sync_copy(acc.at[pl.ds(base, CH)],
                        out_hbm.at[c, pl.ds(base, CH)])


def _pipeline(x1_hbm, srcg_hbm, dstg_hbm, s, sgrps, dgrps, bufs, sems,
              ssems, sem_i, sem_id, acc, goff, ng):
    cpt = ng * GS
    pltpu.sync_copy(srcg_hbm.at[s, goff], sgrps[0])
    pltpu.sync_copy(dstg_hbm.at[s, goff], dgrps[0])
    if ng > 1:
        pltpu.async_copy(srcg_hbm.at[s, goff + 1], sgrps[1], sem_i)
    pltpu.async_copy(x1_hbm.at[sgrps[0].at[0]], bufs[0], sems[0])
    for g in range(cpt):
        grp, row = divmod(g, GS)
        if g + 1 < cpt:
            grp1, row1 = divmod(g + 1, GS)
            if row1 == 0:
                pltpu.make_async_copy(
                    srcg_hbm.at[s, goff + grp1], sgrps[grp1 % 2],
                    sem_i).wait()
            if g >= 1:
                pltpu.make_async_copy(
                    bufs[(g + 1) % 2],
                    acc.at[dgrps[((g - 1) // GS) % 2].at[(g - 1) % GS]],
                    ssems[(g + 1) % 2]).wait()
            pltpu.async_copy(
                x1_hbm.at[sgrps[grp1 % 2].at[row1]],
                bufs[(g + 1) % 2], sems[(g + 1) % 2])
        pltpu.make_async_copy(
            x1_hbm.at[sgrps[grp % 2].at[row]], bufs[g % 2], sems[g % 2]
        ).wait()
        if row == 0:
            if grp > 0:
                pltpu.make_async_copy(
                    dstg_hbm.at[s, goff + grp], dgrps[grp % 2],
                    sem_id).wait()
            if grp + 1 < ng:
                pltpu.async_copy(
                    dstg_hbm.at[s, goff + grp + 1], dgrps[(grp + 1) % 2],
                    sem_id)
        if g + 1 < cpt and (g + 1) % GS == 0 and (g + 1) // GS + 1 < ng:
            pltpu.async_copy(
                srcg_hbm.at[s, goff + (g + 1) // GS + 1],
                sgrps[((g + 1) // GS + 1) % 2], sem_i)
        pltpu.async_copy(bufs[g % 2], acc.at[dgrps[grp % 2].at[row]],
                         ssems[g % 2], add=True)
    pltpu.make_async_copy(
        bufs[(cpt - 2) % 2], acc.at[dgrps[((cpt - 2) // GS) % 2]
                                    .at[(cpt - 2) % GS]],
        ssems[(cpt - 2) % 2]).wait()
    pltpu.make_async_copy(
        bufs[(cpt - 1) % 2], acc.at[dgrps[((cpt - 1) // GS) % 2]
                                    .at[(cpt - 1) % GS]],
        ssems[(cpt - 1) % 2]).wait()


def _normx_body(hists_ref, x_ref, x1_ref):
    deg = jnp.sum(hists_ref[...], axis=0)
    norm = lax.rsqrt(1.0 + deg)[:, None]
    x1_ref[...] = norm * x_ref[...]


@functools.cache
def _sc_kernels():
    cp = pltpu.CompilerParams()
    if "needs_layout_passes" in pltpu.CompilerParams.__dataclass_fields__:
        cp = dataclasses.replace(cp, needs_layout_passes=False)
    deg_k = pl.kernel(
        _deg_body,
        out_type=jax.ShapeDtypeStruct((NC * NS, R), jnp.float32),
        mesh=_sc_mesh(),
        compiler_params=cp,
        scratch_types=[
            pltpu.VMEM((CPT, CH), jnp.int32),
            pltpu.VMEM((R,), jnp.float32),
        ],
    )
    spmm_k = pl.kernel(
        _spmm_body,
        out_type=jax.ShapeDtypeStruct((NC, R, D), jnp.float32),
        mesh=_sc_mesh(),
        scratch_types=[
            pltpu.VMEM((GS, CH), jnp.int32),
            pltpu.VMEM((GS, CH), jnp.int32),
            pltpu.VMEM((GS, CH), jnp.int32),
            pltpu.VMEM((GS, CH), jnp.int32),
            pltpu.VMEM((CH, D), jnp.float32),
            pltpu.VMEM((CH, D), jnp.float32),
            pltpu.VMEM((L, D), jnp.float32),
            pltpu.VMEM_SHARED((R, D), jnp.float32),
            pltpu.SemaphoreType.DMA,
            pltpu.SemaphoreType.DMA,
            pltpu.SemaphoreType.DMA,
            pltpu.SemaphoreType.DMA,
            pltpu.SemaphoreType.DMA,
            pltpu.SemaphoreType.DMA,
        ],
    )
    return deg_k, spmm_k


def _combine_body(hists_ref, aggs_ref, x_ref, o_ref):
    deg = jnp.sum(hists_ref[...], axis=0)
    norm = lax.rsqrt(1.0 + deg)[:, None]
    agg = aggs_ref[0] + aggs_ref[1]
    o_ref[...] = norm * agg + (norm * norm) * x_ref[...]


def kernel(x, edge_index):
    ei = edge_index.astype(jnp.int32)
    dst = ei[0]
    src = ei[1]
    pad = E_PAD - E
    pad_iota = jnp.arange(pad, dtype=jnp.int32)
    dst_flat = jnp.concatenate([dst, TRASH + pad_iota % (R - TRASH)])
    src_flat = jnp.concatenate([src, pad_iota % N])
    dst_deg = dst_flat.reshape(NC, NS, CPT, CH)
    dst_sp = dst_flat.reshape(NS, NGT, GS, CH)
    src_sp = src_flat.reshape(NS, NGT, GS, CH)

    deg_k, spmm_k = _sc_kernels()
    hists = deg_k(dst_deg)

    x1 = pl.pallas_call(
        _normx_body,
        grid=(NBLK,),
        in_specs=[
            pl.BlockSpec((NC * NS, BR), lambda i: (0, i)),
            pl.BlockSpec((BR, D), lambda i: (i, 0)),
        ],
        out_specs=pl.BlockSpec((BR, D), lambda i: (i, 0)),
        out_shape=jax.ShapeDtypeStruct((R, D), jnp.float32),
    )(hists, x)

    aggs = spmm_k(x1, src_sp, dst_sp)

    out = pl.pallas_call(
        _combine_body,
        grid=(NBLK,),
        in_specs=[
            pl.BlockSpec((NC * NS, BR), lambda i: (0, i)),
            pl.BlockSpec((NC, BR, D), lambda i: (0, i, 0)),
            pl.BlockSpec((BR, D), lambda i: (i, 0)),
        ],
        out_specs=pl.BlockSpec((BR, D), lambda i: (i, 0)),
        out_shape=jax.ShapeDtypeStruct((N, D), jnp.float32),
    )(hists, aggs, x)
    return out
